# baseline (speedup 1.0000x reference)
import jax
import jax.numpy as jnp
from jax import lax
from jax.experimental import pallas as pl
from jax.experimental.pallas import tpu as pltpu
from jax.experimental.pallas import tpu_sc as plsc

D_MODEL = 1024
MAX_LEN = 2048
BATCH = 4

NC = 2
NS = 16
NW = NC * NS

T_PER_W = MAX_LEN // NW
C = 16
CHUNKS = T_PER_W // C
GROUPS = C * D_MODEL // 16
JPR = D_MODEL // 16
NBLK = CHUNKS * BATCH


def _pe_add_kernel(x_hbm, pe_hbm, out_hbm,
                   xb0, xb1, yb0, yb1, pb0, pb1,
                   sx0, sx1, sy0, sy1, sp0, sp1):
    wid = lax.axis_index("s") * NC + lax.axis_index("c")
    t_base = wid * T_PER_W

    xb = [xb0, xb1]
    yb = [yb0, yb1]
    pb = [pb0, pb1]
    sx = [sx0, sx1]
    sy = [sy0, sy1]
    sp = [sp0, sp1]

    def xsl(k):
        c, b = divmod(k, BATCH)
        return (b, pl.ds(t_base + c * C, C), slice(None))

    pe_wait = [None] * CHUNKS
    x_wait = [None] * NBLK
    y_wait = [None] * NBLK
    pe_wait[0] = pltpu.async_copy(
        pe_hbm.at[pl.ds(t_base, C), 0, :], pb[0], sp[0])
    x_wait[0] = pltpu.async_copy(x_hbm.at[xsl(0)], xb[0], sx[0])

    def run_fma(xbuf, ybuf, pbuf):
        @plsc.parallel_loop(0, GROUPS, unroll=16)
        def _(i):
            sl = (i // JPR, pl.ds((i % JPR) * 16, 16))
            ybuf[sl] = xbuf[sl] + pbuf[sl] * 0.002

    for k in range(NBLK):
        c, b = divmod(k, BATCH)
        if b == 0 and c + 1 < CHUNKS:
            pe_wait[c + 1] = pltpu.async_copy(
                pe_hbm.at[pl.ds(t_base + (c + 1) * C, C), 0, :],
                pb[(c + 1) % 2], sp[(c + 1) % 2])
        if k + 1 < NBLK:
            x_wait[k + 1] = pltpu.async_copy(
                x_hbm.at[xsl(k + 1)], xb[(k + 1) % 2], sx[(k + 1) % 2])
        if b == 0:
            pe_wait[c].wait()
        x_wait[k].wait()
        if k >= 2:
            y_wait[k - 2].wait()
        run_fma(xb[k % 2], yb[k % 2], pb[c % 2])
        y_wait[k] = pltpu.async_copy(yb[k % 2], out_hbm.at[xsl(k)], sy[k % 2])

    y_wait[NBLK - 2].wait()
    y_wait[NBLK - 1].wait()


@jax.jit
def _pe_add(x, pe):
    mesh = plsc.VectorSubcoreMesh(core_axis_name="c", subcore_axis_name="s")
    return pl.kernel(
        _pe_add_kernel,
        out_type=jax.ShapeDtypeStruct((BATCH, MAX_LEN, D_MODEL), jnp.float32),
        mesh=mesh,
        scratch_types=[pltpu.VMEM((C, D_MODEL), jnp.float32)] * 6
        + [pltpu.SemaphoreType.DMA] * 6,
        compiler_params=pltpu.CompilerParams(use_tc_tiling_on_sc=True, skip_device_barrier=True),
    )(x, pe)


def kernel(x, pe):
    return _pe_add(x, pe)


# speedup vs baseline: 1.0267x; 1.0267x over previous
import jax
import jax.numpy as jnp
from jax import lax
from jax.experimental import pallas as pl
from jax.experimental.pallas import tpu as pltpu
from jax.experimental.pallas import tpu_sc as plsc

D_MODEL = 1024
MAX_LEN = 2048
BATCH = 4

NC = 2
NS = 16
NW = NC * NS

T_PER_W = MAX_LEN // NW
C = 16
CHUNKS = T_PER_W // C
GROUPS = C * D_MODEL // 16
JPR = D_MODEL // 16
NBLK = CHUNKS * BATCH


def _pe_add_kernel(x_hbm, pe_hbm, out_hbm,
                   xb0, xb1, yb0, yb1, pb0, pb1,
                   sx0, sx1, sy0, sy1, sp0, sp1):
    wid = lax.axis_index("s") * NC + lax.axis_index("c")
    t_base = wid * T_PER_W

    xb = [xb0, xb1]
    yb = [yb0, yb1]
    pb = [pb0, pb1]
    sx = [sx0, sx1]
    sy = [sy0, sy1]
    sp = [sp0, sp1]

    def xsl(k):
        c, b = divmod(k, BATCH)
        return (b, pl.ds(t_base + c * C, C), slice(None))

    pe_wait = [None] * CHUNKS
    x_wait = [None] * NBLK
    y_wait = [None] * NBLK
    pe_wait[0] = pltpu.async_copy(
        pe_hbm.at[pl.ds(t_base, C), 0, :], pb[0], sp[0])
    x_wait[0] = pltpu.async_copy(x_hbm.at[xsl(0)], xb[0], sx[0])

    def run_fma(xbuf, ybuf, pbuf):
        @plsc.parallel_loop(0, GROUPS, unroll=8)
        def _(i):
            sl = (i // JPR, pl.ds((i % JPR) * 16, 16))
            ybuf[sl] = xbuf[sl] + pbuf[sl] * 0.002

    for k in range(NBLK):
        c, b = divmod(k, BATCH)
        if b == 0 and c + 1 < CHUNKS:
            pe_wait[c + 1] = pltpu.async_copy(
                pe_hbm.at[pl.ds(t_base + (c + 1) * C, C), 0, :],
                pb[(c + 1) % 2], sp[(c + 1) % 2])
        if k + 1 < NBLK:
            x_wait[k + 1] = pltpu.async_copy(
                x_hbm.at[xsl(k + 1)], xb[(k + 1) % 2], sx[(k + 1) % 2])
        if b == 0:
            pe_wait[c].wait()
        x_wait[k].wait()
        if k >= 2:
            y_wait[k - 2].wait()
        run_fma(xb[k % 2], yb[k % 2], pb[c % 2])
        y_wait[k] = pltpu.async_copy(yb[k % 2], out_hbm.at[xsl(k)], sy[k % 2])

    y_wait[NBLK - 2].wait()
    y_wait[NBLK - 1].wait()


@jax.jit
def _pe_add(x, pe):
    mesh = plsc.VectorSubcoreMesh(core_axis_name="c", subcore_axis_name="s")
    return pl.kernel(
        _pe_add_kernel,
        out_type=jax.ShapeDtypeStruct((BATCH, MAX_LEN, D_MODEL), jnp.float32),
        mesh=mesh,
        scratch_types=[pltpu.VMEM((C, D_MODEL), jnp.float32)] * 6
        + [pltpu.SemaphoreType.DMA] * 6,
        compiler_params=pltpu.CompilerParams(use_tc_tiling_on_sc=True),
    )(x, pe)


def kernel(x, pe):
    return _pe_add(x, pe)
